# baseline (speedup 1.0000x reference)
import functools

import jax
import jax.numpy as jnp
from jax import lax
from jax.experimental import pallas as pl
from jax.experimental.pallas import tpu as pltpu
from jax.experimental.pallas import tpu_sc as plsc

_ORDERS = (2, 3, 4)


def _prep_body(ids_ref, idx_ref, *, buckets, num_heads):
    ids = ids_ref[...]
    h = ids
    b, s = ids.shape
    for i in range(num_heads):
        order = _ORDERS[i]
        start = 1 if i == 0 else _ORDERS[i - 1]
        for j in range(start, order):
            shifted = jnp.concatenate(
                [jnp.zeros((b, j), jnp.int32), ids[:, : s - j]], axis=1
            )
            h = jnp.bitwise_xor(h, shifted)
        hid = jnp.bitwise_and(h, buckets - 1) + i * buckets
        for bb in range(b):
            idx_ref[pl.ds((i * b + bb) * s, s)] = hid[bb]


def _prep(ids, num_heads, buckets):
    b, s = ids.shape
    return pl.pallas_call(
        functools.partial(_prep_body, buckets=buckets, num_heads=num_heads),
        out_shape=jax.ShapeDtypeStruct((num_heads * b * s,), jnp.int32),
    )(ids)


def _make_sc_gather(num_rows, hash_dim, num_idx):
    info = plsc.get_sparse_core_info()
    nc, ns = info.num_cores, info.num_subcores
    nw = nc * ns
    per_w = num_idx // nw
    assert per_w * nw == num_idx
    chunk = 128
    n_chunks = per_w // chunk
    assert n_chunks * chunk == per_w

    mesh = plsc.VectorSubcoreMesh(core_axis_name="c", subcore_axis_name="s")

    @functools.partial(
        pl.kernel,
        mesh=mesh,
        out_type=jax.ShapeDtypeStruct((num_idx, hash_dim), jnp.float32),
        scratch_types=[
            pltpu.VMEM((per_w,), jnp.int32),
            pltpu.VMEM((2, chunk, hash_dim), jnp.float32),
            pltpu.SemaphoreType.DMA,
            pltpu.SemaphoreType.DMA,
            pltpu.SemaphoreType.DMA,
            pltpu.SemaphoreType.DMA,
        ],
    )
    def gather_k(table_hbm, idx_hbm, out_hbm, idx_v, rows_v, g0, g1, o0, o1):
        wid = lax.axis_index("s") * nc + lax.axis_index("c")
        base = wid * per_w
        gsem = (g0, g1)
        osem = (o0, o1)
        pltpu.sync_copy(idx_hbm.at[pl.ds(base, per_w)], idx_v)

        def start_gather(j):
            b = j % 2
            return pltpu.async_copy(
                table_hbm.at[idx_v.at[pl.ds(j * chunk, chunk)]],
                rows_v.at[b],
                gsem[b],
            )

        def start_out(j):
            b = j % 2
            return pltpu.async_copy(
                rows_v.at[b], out_hbm.at[pl.ds(base + j * chunk, chunk)], osem[b]
            )

        ocp = [None, None]
        gcp = [None, None]
        gcp[0] = start_gather(0)
        for j in range(n_chunks):
            b = j % 2
            nb = (j + 1) % 2
            if j + 1 < n_chunks:
                if ocp[nb] is not None:
                    ocp[nb].wait()
                gcp[nb] = start_gather(j + 1)
            gcp[b].wait()
            ocp[b] = start_out(j)
        ocp[(n_chunks - 1) % 2].wait()
        if ocp[n_chunks % 2] is not None:
            ocp[n_chunks % 2].wait()

    return gather_k


def _gates_body(hid_ref, wg_ref, bg_ref, g_ref):
    x = hid_ref[...]
    xwg = lax.dot_general(
        x,
        wg_ref[...],
        dimension_numbers=(((1,), (1,)), ((), ())),
        preferred_element_type=jnp.float32,
    )
    g_ref[...] = jax.nn.sigmoid(xwg + bg_ref[...])


def _gates(hid2, wg, bg, blk, k2):
    bs, d = hid2.shape
    nh = wg.shape[0]
    nblk = bs // blk
    return pl.pallas_call(
        _gates_body,
        grid=(nblk - k2,),
        in_specs=[
            pl.BlockSpec((blk, d), lambda t: (t + k2, 0)),
            pl.BlockSpec((nh, d), lambda t: (0, 0)),
            pl.BlockSpec((1, nh), lambda t: (0, 0)),
        ],
        out_specs=pl.BlockSpec((blk, nh), lambda t: (t + k2, 0)),
        out_shape=jax.ShapeDtypeStruct((bs, nh), jnp.float32),
    )(hid2, wg, bg)


def _final_body(hid_ref, parts_ref, g1_ref, wg_ref, bg_ref, wo_ref, bo_ref,
                out_ref, *, k2):
    t = pl.program_id(0)
    nh = parts_ref.shape[0]
    hd = parts_ref.shape[2]
    x = hid_ref[...]
    gf = jax.nn.sigmoid(
        lax.dot_general(
            x,
            wg_ref[...],
            dimension_numbers=(((1,), (1,)), ((), ())),
            preferred_element_type=jnp.float32,
        )
        + bg_ref[...]
    )
    g = jnp.where(t < k2, gf, g1_ref[...]).astype(jnp.bfloat16)
    acc = None
    for i in range(nh):
        p = parts_ref[i].astype(jnp.bfloat16)
        gp = p * g[:, i : i + 1]
        contrib = lax.dot_general(
            gp,
            wo_ref[:, i * hd : (i + 1) * hd].astype(jnp.bfloat16),
            dimension_numbers=(((1,), (1,)), ((), ())),
            preferred_element_type=jnp.float32,
        )
        acc = contrib if acc is None else acc + contrib
    out_ref[...] = acc + bo_ref[...]


def _final(hid2, parts3, g, wg, bg, wo, bo, blk, k2):
    nh, bs, hd = parts3.shape
    d = wo.shape[0]
    return pl.pallas_call(
        functools.partial(_final_body, k2=k2),
        grid=(bs // blk,),
        in_specs=[
            pl.BlockSpec((blk, d), lambda t: (jnp.minimum(t, k2 - 1), 0)),
            pl.BlockSpec((nh, blk, hd), lambda t: (0, t, 0)),
            pl.BlockSpec((blk, nh), lambda t: (jnp.maximum(t, k2), 0)),
            pl.BlockSpec((nh, d), lambda t: (0, 0)),
            pl.BlockSpec((1, nh), lambda t: (0, 0)),
            pl.BlockSpec(wo.shape, lambda t: (0, 0)),
            pl.BlockSpec((1, d), lambda t: (0, 0)),
        ],
        out_specs=pl.BlockSpec((blk, d), lambda t: (t, 0)),
        out_shape=jax.ShapeDtypeStruct((bs, d), jnp.float32),
    )(hid2, parts3, g, wg, bg, wo, bo)


def kernel(input_ids, hidden_state, tables, W_gate, b_gate, W_out, b_out):
    b, s = input_ids.shape
    nh, buckets, hd = tables.shape
    d = hidden_state.shape[-1]
    bs = b * s

    ids = input_ids.astype(jnp.int32)
    idx_flat = _prep(ids, nh, buckets)

    tables_flat = tables.reshape(nh * buckets, hd)
    parts = _make_sc_gather(nh * buckets, hd, nh * bs)(tables_flat, idx_flat)
    parts3 = parts.reshape(nh, bs, hd)

    hid2 = hidden_state.reshape(bs, d)
    bg = b_gate.reshape(1, nh)
    bo = b_out.reshape(1, d)

    blk = 1024
    k2 = 6
    g = _gates(hid2, W_gate, bg, blk, k2)
    out = _final(hid2, parts3, g, W_gate, bg, W_out, bo, blk, k2)
    return out.reshape(b, s, d)


# speedup vs baseline: 1.1425x; 1.1425x over previous
import functools

import jax
import jax.numpy as jnp
from jax import lax
from jax.experimental import pallas as pl
from jax.experimental.pallas import tpu as pltpu
from jax.experimental.pallas import tpu_sc as plsc

_ORDERS = (2, 3, 4)


def _prep_body(ids_ref, idx_ref, *, buckets, num_heads):
    ids = ids_ref[...]
    h = ids
    b, s = ids.shape
    for i in range(num_heads):
        order = _ORDERS[i]
        start = 1 if i == 0 else _ORDERS[i - 1]
        for j in range(start, order):
            shifted = jnp.concatenate(
                [jnp.zeros((b, j), jnp.int32), ids[:, : s - j]], axis=1
            )
            h = jnp.bitwise_xor(h, shifted)
        hid = jnp.bitwise_and(h, buckets - 1) + i * buckets
        for bb in range(b):
            idx_ref[pl.ds((i * b + bb) * s, s)] = hid[bb]


def _prep(ids, num_heads, buckets):
    b, s = ids.shape
    return pl.pallas_call(
        functools.partial(_prep_body, buckets=buckets, num_heads=num_heads),
        out_shape=jax.ShapeDtypeStruct((num_heads * b * s,), jnp.int32),
    )(ids)


def _make_sc_gather(num_rows, hash_dim, num_idx, half):
    info = plsc.get_sparse_core_info()
    nc, ns = info.num_cores, info.num_subcores
    nw = nc * ns
    pairs = num_idx // 2
    per_w = pairs // nw
    assert per_w * nw == pairs
    chunk = 64
    n_chunks = per_w // chunk
    assert n_chunks * chunk == per_w
    assert half % chunk == 0

    mesh = plsc.VectorSubcoreMesh(core_axis_name="c", subcore_axis_name="s")

    @functools.partial(
        pl.kernel,
        mesh=mesh,
        out_type=jax.ShapeDtypeStruct((pairs, hash_dim), jnp.uint32),
        scratch_types=[
            pltpu.VMEM((2, 2, chunk), jnp.int32),
            pltpu.VMEM((2, 2, chunk, hash_dim), jnp.uint32),
            pltpu.VMEM((2, chunk, hash_dim), jnp.uint32),
            pltpu.SemaphoreType.DMA,
            pltpu.SemaphoreType.DMA,
            pltpu.SemaphoreType.DMA,
            pltpu.SemaphoreType.DMA,
            pltpu.SemaphoreType.DMA,
            pltpu.SemaphoreType.DMA,
        ],
    )
    def gather_k(table_hbm, idx_hbm, out_hbm, idx_v, rows_v, pk_v,
                 ia0, ia1, g0, g1, o0, o1):
        wid = lax.axis_index("s") * nc + lax.axis_index("c")
        base = wid * per_w
        isem = (ia0, ia1)
        gsem = (g0, g1)
        osem = (o0, o1)
        half16 = jnp.full((16,), 0x8000, jnp.uint32)
        himask = jnp.full((16,), 0xFFFF0000, jnp.uint32)
        sh16 = jnp.full((16,), 16, jnp.uint32)

        def start_idx(j):
            b = j % 2
            p0 = base + j * chunk
            a0 = p0 + (p0 // half) * half
            c1 = pltpu.async_copy(idx_hbm.at[pl.ds(a0, chunk)], idx_v.at[b, 0],
                                  isem[b])
            c2 = pltpu.async_copy(idx_hbm.at[pl.ds(a0 + half, chunk)],
                                  idx_v.at[b, 1], isem[b])
            return (c1, c2)

        def start_gather(j):
            b = j % 2
            c1 = pltpu.async_copy(table_hbm.at[idx_v.at[b, 0]], rows_v.at[b, 0],
                                  gsem[b])
            c2 = pltpu.async_copy(table_hbm.at[idx_v.at[b, 1]], rows_v.at[b, 1],
                                  gsem[b])
            return (c1, c2)

        def start_out(j):
            b = j % 2
            return pltpu.async_copy(
                pk_v.at[b], out_hbm.at[pl.ds(base + j * chunk, chunk)], osem[b]
            )

        def pack_chunk(b):
            def body(q, c):
                for m in range(hash_dim // 16):
                    av = rows_v[b, 0, q, pl.ds(16 * m, 16)]
                    bv = rows_v[b, 1, q, pl.ds(16 * m, 16)]
                    w = ((av + half16) >> sh16) | ((bv + half16) & himask)
                    pk_v[b, q, pl.ds(16 * m, 16)] = w
                return c

            lax.fori_loop(0, chunk, body, 0)

        icp = [None, None]
        gcp = [None, None]
        ocp = [None, None]
        icp[0] = start_idx(0)
        icp[0][0].wait()
        icp[0][1].wait()
        icp[1] = start_idx(1) if n_chunks > 1 else None
        gcp[0] = start_gather(0)
        for j in range(n_chunks):
            b = j % 2
            nb = (j + 1) % 2
            if j + 1 < n_chunks:
                icp[nb][0].wait()
                icp[nb][1].wait()
                gcp[nb] = start_gather(j + 1)
            gcp[b][0].wait()
            gcp[b][1].wait()
            if j + 2 < n_chunks:
                icp[b] = start_idx(j + 2)
            if ocp[b] is not None:
                ocp[b].wait()
            pack_chunk(b)
            ocp[b] = start_out(j)
        ocp[(n_chunks - 1) % 2].wait()
        if ocp[n_chunks % 2] is not None:
            ocp[n_chunks % 2].wait()

    return gather_k


def _gates_body(hid_ref, wg_ref, bg_ref, g_ref):
    x = hid_ref[...]
    xwg = lax.dot_general(
        x,
        wg_ref[...],
        dimension_numbers=(((1,), (1,)), ((), ())),
        preferred_element_type=jnp.float32,
    )
    g_ref[...] = jax.nn.sigmoid(xwg + bg_ref[...])


def _gates(hid2, wg, bg, blk=2048):
    bs, d = hid2.shape
    nh = wg.shape[0]
    return pl.pallas_call(
        _gates_body,
        grid=(bs // blk,),
        in_specs=[
            pl.BlockSpec((blk, d), lambda t: (t, 0)),
            pl.BlockSpec((nh, d), lambda t: (0, 0)),
            pl.BlockSpec((1, nh), lambda t: (0, 0)),
        ],
        out_specs=pl.BlockSpec((blk, nh), lambda t: (t, 0)),
        out_shape=jax.ShapeDtypeStruct((bs, nh), jnp.float32),
    )(hid2, wg, bg)


def _final_body(parts_ref, ga_ref, gb_ref, wo_ref, bo_ref, out_ref):
    ga = ga_ref[...].astype(jnp.bfloat16)
    gb = gb_ref[...].astype(jnp.bfloat16)
    nh = parts_ref.shape[0]
    hd = parts_ref.shape[2]
    acc_a = None
    acc_b = None
    for i in range(nh):
        u = parts_ref[i]
        pa = lax.bitcast_convert_type(u << 16, jnp.float32).astype(jnp.bfloat16)
        pb = lax.bitcast_convert_type(
            u & jnp.uint32(0xFFFF0000), jnp.float32
        ).astype(jnp.bfloat16)
        wo_i = wo_ref[:, i * hd : (i + 1) * hd].astype(jnp.bfloat16)
        dn = (((1,), (1,)), ((), ()))
        ca = lax.dot_general(pa * ga[:, i : i + 1], wo_i, dimension_numbers=dn,
                             preferred_element_type=jnp.float32)
        cb = lax.dot_general(pb * gb[:, i : i + 1], wo_i, dimension_numbers=dn,
                             preferred_element_type=jnp.float32)
        acc_a = ca if acc_a is None else acc_a + ca
        acc_b = cb if acc_b is None else acc_b + cb
    out_ref[0] = acc_a + bo_ref[...]
    out_ref[1] = acc_b + bo_ref[...]


def _final(parts3, g, wo, bo, blkp=512):
    nh, half, hd = parts3.shape
    d = wo.shape[0]
    nblk = half // blkp
    return pl.pallas_call(
        _final_body,
        grid=(nblk,),
        in_specs=[
            pl.BlockSpec((nh, blkp, hd), lambda t: (0, t, 0)),
            pl.BlockSpec((blkp, g.shape[1]), lambda t: (t, 0)),
            pl.BlockSpec((blkp, g.shape[1]), lambda t, n=nblk: (t + n, 0)),
            pl.BlockSpec(wo.shape, lambda t: (0, 0)),
            pl.BlockSpec((1, d), lambda t: (0, 0)),
        ],
        out_specs=pl.BlockSpec((2, blkp, d), lambda t: (0, t, 0)),
        out_shape=jax.ShapeDtypeStruct((2, half, d), jnp.float32),
    )(parts3, g, g, wo, bo)


def kernel(input_ids, hidden_state, tables, W_gate, b_gate, W_out, b_out):
    b, s = input_ids.shape
    nh, buckets, hd = tables.shape
    d = hidden_state.shape[-1]
    bs = b * s

    ids = input_ids.astype(jnp.int32)
    idx_flat = _prep(ids, nh, buckets)

    half = bs // 2
    tables_u = jax.lax.bitcast_convert_type(
        tables.reshape(nh * buckets, hd), jnp.uint32
    )
    parts = _make_sc_gather(nh * buckets, hd, nh * bs, half)(tables_u, idx_flat)
    parts3 = parts.reshape(nh, half, hd)

    hid2 = hidden_state.reshape(bs, d)
    bg = b_gate.reshape(1, nh)
    bo = b_out.reshape(1, d)

    g = _gates(hid2, W_gate, bg)
    out = _final(parts3, g, W_out, bo)
    return out.reshape(b, s, d)


# speedup vs baseline: 1.1633x; 1.0182x over previous
import functools

import jax
import jax.numpy as jnp
from jax import lax
from jax.experimental import pallas as pl
from jax.experimental.pallas import tpu as pltpu
from jax.experimental.pallas import tpu_sc as plsc

_ORDERS = (2, 3, 4)


def _prep_body(ids_ref, tbl_ref, idx_ref, tblu_ref, *, buckets, num_heads):
    ids = ids_ref[...]
    h = ids
    b, s = ids.shape
    for i in range(num_heads):
        order = _ORDERS[i]
        start = 1 if i == 0 else _ORDERS[i - 1]
        for j in range(start, order):
            shifted = jnp.concatenate(
                [jnp.zeros((b, j), jnp.int32), ids[:, : s - j]], axis=1
            )
            h = jnp.bitwise_xor(h, shifted)
        hid = jnp.bitwise_and(h, buckets - 1) + i * buckets
        for bb in range(b):
            idx_ref[pl.ds((i * b + bb) * s, s)] = hid[bb]
    tblu_ref[...] = lax.bitcast_convert_type(tbl_ref[...], jnp.uint32)


def _prep(ids, tables_flat, num_heads, buckets):
    b, s = ids.shape
    return pl.pallas_call(
        functools.partial(_prep_body, buckets=buckets, num_heads=num_heads),
        out_shape=[
            jax.ShapeDtypeStruct((num_heads * b * s,), jnp.int32),
            jax.ShapeDtypeStruct(tables_flat.shape, jnp.uint32),
        ],
    )(ids, tables_flat)


def _make_sc_gather(num_rows, hash_dim, num_idx, half):
    info = plsc.get_sparse_core_info()
    nc, ns = info.num_cores, info.num_subcores
    nw = nc * ns
    pairs = num_idx // 2
    per_w = pairs // nw
    assert per_w * nw == pairs
    chunk = 64
    n_chunks = per_w // chunk
    assert n_chunks * chunk == per_w
    assert half % chunk == 0

    mesh = plsc.VectorSubcoreMesh(core_axis_name="c", subcore_axis_name="s")

    @functools.partial(
        pl.kernel,
        mesh=mesh,
        out_type=jax.ShapeDtypeStruct((pairs, hash_dim), jnp.uint32),
        scratch_types=[
            pltpu.VMEM((2, 2, chunk), jnp.int32),
            pltpu.VMEM((2, 2, chunk, hash_dim), jnp.uint32),
            pltpu.VMEM((2, chunk, hash_dim), jnp.uint32),
            pltpu.SemaphoreType.DMA,
            pltpu.SemaphoreType.DMA,
            pltpu.SemaphoreType.DMA,
            pltpu.SemaphoreType.DMA,
            pltpu.SemaphoreType.DMA,
            pltpu.SemaphoreType.DMA,
        ],
    )
    def gather_k(table_hbm, idx_hbm, out_hbm, idx_v, rows_v, pk_v,
                 ia0, ia1, g0, g1, o0, o1):
        wid = lax.axis_index("s") * nc + lax.axis_index("c")
        base = wid * per_w
        isem = (ia0, ia1)
        gsem = (g0, g1)
        osem = (o0, o1)
        half16 = jnp.full((16,), 0x8000, jnp.uint32)
        himask = jnp.full((16,), 0xFFFF0000, jnp.uint32)
        sh16 = jnp.full((16,), 16, jnp.uint32)

        def start_idx(j):
            b = j % 2
            p0 = base + j * chunk
            a0 = p0 + (p0 // half) * half
            c1 = pltpu.async_copy(idx_hbm.at[pl.ds(a0, chunk)], idx_v.at[b, 0],
                                  isem[b])
            c2 = pltpu.async_copy(idx_hbm.at[pl.ds(a0 + half, chunk)],
                                  idx_v.at[b, 1], isem[b])
            return (c1, c2)

        def start_gather(j):
            b = j % 2
            c1 = pltpu.async_copy(table_hbm.at[idx_v.at[b, 0]], rows_v.at[b, 0],
                                  gsem[b])
            c2 = pltpu.async_copy(table_hbm.at[idx_v.at[b, 1]], rows_v.at[b, 1],
                                  gsem[b])
            return (c1, c2)

        def start_out(j):
            b = j % 2
            return pltpu.async_copy(
                pk_v.at[b], out_hbm.at[pl.ds(base + j * chunk, chunk)], osem[b]
            )

        def pack_chunk(b):
            def body(q, c):
                for m in range(hash_dim // 16):
                    av = rows_v[b, 0, q, pl.ds(16 * m, 16)]
                    bv = rows_v[b, 1, q, pl.ds(16 * m, 16)]
                    w = ((av + half16) >> sh16) | ((bv + half16) & himask)
                    pk_v[b, q, pl.ds(16 * m, 16)] = w
                return c

            lax.fori_loop(0, chunk, body, 0)

        icp = [None, None]
        gcp = [None, None]
        ocp = [None, None]
        icp[0] = start_idx(0)
        icp[0][0].wait()
        icp[0][1].wait()
        icp[1] = start_idx(1) if n_chunks > 1 else None
        gcp[0] = start_gather(0)
        for j in range(n_chunks):
            b = j % 2
            nb = (j + 1) % 2
            if j + 1 < n_chunks:
                icp[nb][0].wait()
                icp[nb][1].wait()
                gcp[nb] = start_gather(j + 1)
            gcp[b][0].wait()
            gcp[b][1].wait()
            if j + 2 < n_chunks:
                icp[b] = start_idx(j + 2)
            if ocp[b] is not None:
                ocp[b].wait()
            pack_chunk(b)
            ocp[b] = start_out(j)
        ocp[(n_chunks - 1) % 2].wait()
        if ocp[n_chunks % 2] is not None:
            ocp[n_chunks % 2].wait()

    return gather_k


def _gates_body(hid_ref, wg_ref, bg_ref, g_ref):
    x = hid_ref[...]
    xwg = lax.dot_general(
        x,
        wg_ref[...],
        dimension_numbers=(((1,), (1,)), ((), ())),
        preferred_element_type=jnp.float32,
    )
    g_ref[...] = jax.nn.sigmoid(xwg + bg_ref[...])


def _gates(hid2, wg, bg, blk=2048):
    bs, d = hid2.shape
    nh = wg.shape[0]
    return pl.pallas_call(
        _gates_body,
        grid=(bs // blk,),
        in_specs=[
            pl.BlockSpec((blk, d), lambda t: (t, 0)),
            pl.BlockSpec((nh, d), lambda t: (0, 0)),
            pl.BlockSpec((1, nh), lambda t: (0, 0)),
        ],
        out_specs=pl.BlockSpec((blk, nh), lambda t: (t, 0)),
        out_shape=jax.ShapeDtypeStruct((bs, nh), jnp.float32),
    )(hid2, wg, bg)


def _final_body(parts_ref, ga_ref, gb_ref, wo_ref, bo_ref, out_ref):
    ga = ga_ref[...].astype(jnp.bfloat16)
    gb = gb_ref[...].astype(jnp.bfloat16)
    nh = parts_ref.shape[0]
    hd = parts_ref.shape[2]
    acc_a = None
    acc_b = None
    for i in range(nh):
        u = parts_ref[i]
        pa = lax.bitcast_convert_type(u << 16, jnp.float32).astype(jnp.bfloat16)
        pb = lax.bitcast_convert_type(
            u & jnp.uint32(0xFFFF0000), jnp.float32
        ).astype(jnp.bfloat16)
        wo_i = wo_ref[:, i * hd : (i + 1) * hd].astype(jnp.bfloat16)
        dn = (((1,), (1,)), ((), ()))
        ca = lax.dot_general(pa * ga[:, i : i + 1], wo_i, dimension_numbers=dn,
                             preferred_element_type=jnp.float32)
        cb = lax.dot_general(pb * gb[:, i : i + 1], wo_i, dimension_numbers=dn,
                             preferred_element_type=jnp.float32)
        acc_a = ca if acc_a is None else acc_a + ca
        acc_b = cb if acc_b is None else acc_b + cb
    out_ref[0] = acc_a + bo_ref[...]
    out_ref[1] = acc_b + bo_ref[...]


def _final(parts3, g, wo, bo, blkp=512):
    nh, half, hd = parts3.shape
    d = wo.shape[0]
    nblk = half // blkp
    return pl.pallas_call(
        _final_body,
        grid=(nblk,),
        in_specs=[
            pl.BlockSpec((nh, blkp, hd), lambda t: (0, t, 0)),
            pl.BlockSpec((blkp, g.shape[1]), lambda t: (t, 0)),
            pl.BlockSpec((blkp, g.shape[1]), lambda t, n=nblk: (t + n, 0)),
            pl.BlockSpec(wo.shape, lambda t: (0, 0)),
            pl.BlockSpec((1, d), lambda t: (0, 0)),
        ],
        out_specs=pl.BlockSpec((2, blkp, d), lambda t: (0, t, 0)),
        out_shape=jax.ShapeDtypeStruct((2, half, d), jnp.float32),
    )(parts3, g, g, wo, bo)


def kernel(input_ids, hidden_state, tables, W_gate, b_gate, W_out, b_out):
    b, s = input_ids.shape
    nh, buckets, hd = tables.shape
    d = hidden_state.shape[-1]
    bs = b * s

    ids = input_ids.astype(jnp.int32)
    idx_flat, tables_u = _prep(ids, tables.reshape(nh * buckets, hd), nh, buckets)

    half = bs // 2
    parts = _make_sc_gather(nh * buckets, hd, nh * bs, half)(tables_u, idx_flat)
    parts3 = parts.reshape(nh, half, hd)

    hid2 = hidden_state.reshape(bs, d)
    bg = b_gate.reshape(1, nh)
    bo = b_out.reshape(1, d)

    g = _gates(hid2, W_gate, bg)
    out = _final(parts3, g, W_out, bo)
    return out.reshape(b, s, d)


# speedup vs baseline: 1.2111x; 1.0412x over previous
import functools

import jax
import jax.numpy as jnp
from jax import lax
from jax.experimental import pallas as pl
from jax.experimental.pallas import tpu as pltpu
from jax.experimental.pallas import tpu_sc as plsc

_ORDERS = (2, 3, 4)


def _prep_body(ids_ref, tbl_ref, idx_ref, tblu_ref, *, buckets, num_heads):
    ids = ids_ref[...]
    h = ids
    b, s = ids.shape
    for i in range(num_heads):
        order = _ORDERS[i]
        start = 1 if i == 0 else _ORDERS[i - 1]
        for j in range(start, order):
            shifted = jnp.concatenate(
                [jnp.zeros((b, j), jnp.int32), ids[:, : s - j]], axis=1
            )
            h = jnp.bitwise_xor(h, shifted)
        hid = jnp.bitwise_and(h, buckets - 1) + i * buckets
        for bb in range(b):
            idx_ref[pl.ds((i * b + bb) * s, s)] = hid[bb]
    tblu_ref[...] = lax.bitcast_convert_type(tbl_ref[...], jnp.uint32)


def _prep(ids, tables_flat, num_heads, buckets):
    b, s = ids.shape
    return pl.pallas_call(
        functools.partial(_prep_body, buckets=buckets, num_heads=num_heads),
        out_shape=[
            jax.ShapeDtypeStruct((num_heads * b * s,), jnp.int32),
            jax.ShapeDtypeStruct(tables_flat.shape, jnp.uint32),
        ],
    )(ids, tables_flat)


def _make_sc_gather(num_rows, hash_dim, num_idx, half):
    info = plsc.get_sparse_core_info()
    nc, ns = info.num_cores, info.num_subcores
    nw = nc * ns
    pairs = num_idx // 2
    per_w = pairs // nw
    assert per_w * nw == pairs
    chunk = 64
    n_chunks = per_w // chunk
    assert n_chunks * chunk == per_w
    assert half % chunk == 0

    mesh = plsc.VectorSubcoreMesh(core_axis_name="c", subcore_axis_name="s")

    @functools.partial(
        pl.kernel,
        mesh=mesh,
        out_type=jax.ShapeDtypeStruct((pairs, hash_dim), jnp.uint32),
        scratch_types=[
            pltpu.VMEM((2, 2, chunk), jnp.int32),
            pltpu.VMEM((2, 2, chunk, hash_dim), jnp.uint32),
            pltpu.VMEM((2, chunk, hash_dim), jnp.uint32),
            pltpu.SemaphoreType.DMA,
            pltpu.SemaphoreType.DMA,
            pltpu.SemaphoreType.DMA,
            pltpu.SemaphoreType.DMA,
            pltpu.SemaphoreType.DMA,
            pltpu.SemaphoreType.DMA,
        ],
    )
    def gather_k(table_hbm, idx_hbm, out_hbm, idx_v, rows_v, pk_v,
                 ia0, ia1, g0, g1, o0, o1):
        wid = lax.axis_index("s") * nc + lax.axis_index("c")
        base = wid * per_w
        isem = (ia0, ia1)
        gsem = (g0, g1)
        osem = (o0, o1)
        half16 = jnp.full((16,), 0x8000, jnp.uint32)
        himask = jnp.full((16,), 0xFFFF0000, jnp.uint32)
        sh16 = jnp.full((16,), 16, jnp.uint32)

        def start_idx(j):
            b = j % 2
            p0 = base + j * chunk
            a0 = p0 + (p0 // half) * half
            c1 = pltpu.async_copy(idx_hbm.at[pl.ds(a0, chunk)], idx_v.at[b, 0],
                                  isem[b])
            c2 = pltpu.async_copy(idx_hbm.at[pl.ds(a0 + half, chunk)],
                                  idx_v.at[b, 1], isem[b])
            return (c1, c2)

        def start_gather(j):
            b = j % 2
            c1 = pltpu.async_copy(table_hbm.at[idx_v.at[b, 0]], rows_v.at[b, 0],
                                  gsem[b])
            c2 = pltpu.async_copy(table_hbm.at[idx_v.at[b, 1]], rows_v.at[b, 1],
                                  gsem[b])
            return (c1, c2)

        def start_out(j):
            b = j % 2
            return pltpu.async_copy(
                pk_v.at[b], out_hbm.at[pl.ds(base + j * chunk, chunk)], osem[b]
            )

        def pack_chunk(b):
            def body(q, c):
                for m in range(hash_dim // 16):
                    av = rows_v[b, 0, q, pl.ds(16 * m, 16)]
                    bv = rows_v[b, 1, q, pl.ds(16 * m, 16)]
                    w = ((av + half16) >> sh16) | ((bv + half16) & himask)
                    pk_v[b, q, pl.ds(16 * m, 16)] = w
                return c

            lax.fori_loop(0, chunk, body, 0)

        icp = [None, None]
        gcp = [None, None]
        ocp = [None, None]
        icp[0] = start_idx(0)
        icp[0][0].wait()
        icp[0][1].wait()
        icp[1] = start_idx(1) if n_chunks > 1 else None
        gcp[0] = start_gather(0)
        for j in range(n_chunks):
            b = j % 2
            nb = (j + 1) % 2
            if j + 1 < n_chunks:
                icp[nb][0].wait()
                icp[nb][1].wait()
                gcp[nb] = start_gather(j + 1)
            gcp[b][0].wait()
            gcp[b][1].wait()
            if j + 2 < n_chunks:
                icp[b] = start_idx(j + 2)
            if ocp[b] is not None:
                ocp[b].wait()
            pack_chunk(b)
            ocp[b] = start_out(j)
        ocp[(n_chunks - 1) % 2].wait()
        if ocp[n_chunks % 2] is not None:
            ocp[n_chunks % 2].wait()

    return gather_k


def _gates_body(hid_ref, wg_ref, bg_ref, g_ref):
    x = hid_ref[...]
    xwg = lax.dot_general(
        x,
        wg_ref[...],
        dimension_numbers=(((1,), (1,)), ((), ())),
        preferred_element_type=jnp.float32,
    )
    g_ref[...] = jax.nn.sigmoid(xwg + bg_ref[...])


def _gates(hid2, wg, bg, blk=2048):
    bs, d = hid2.shape
    nh = wg.shape[0]
    return pl.pallas_call(
        _gates_body,
        grid=(bs // blk,),
        in_specs=[
            pl.BlockSpec((blk, d), lambda t: (t, 0)),
            pl.BlockSpec((nh, d), lambda t: (0, 0)),
            pl.BlockSpec((1, nh), lambda t: (0, 0)),
        ],
        out_specs=pl.BlockSpec((blk, nh), lambda t: (t, 0)),
        out_shape=jax.ShapeDtypeStruct((bs, nh), jnp.float32),
    )(hid2, wg, bg)


def _final_body(parts_ref, ga_ref, gb_ref, wo_ref, bo_ref, out_ref):
    ga = ga_ref[...].astype(jnp.bfloat16)
    gb = gb_ref[...].astype(jnp.bfloat16)
    nh = parts_ref.shape[0]
    hd = parts_ref.shape[2]
    acc_a = None
    acc_b = None
    for i in range(nh):
        u = parts_ref[i]
        pa = lax.bitcast_convert_type(u << 16, jnp.float32).astype(jnp.bfloat16)
        pb = lax.bitcast_convert_type(
            u & jnp.uint32(0xFFFF0000), jnp.float32
        ).astype(jnp.bfloat16)
        wo_i = wo_ref[:, i * hd : (i + 1) * hd].astype(jnp.bfloat16)
        dn = (((1,), (1,)), ((), ()))
        ca = lax.dot_general(pa * ga[:, i : i + 1], wo_i, dimension_numbers=dn,
                             preferred_element_type=jnp.float32)
        cb = lax.dot_general(pb * gb[:, i : i + 1], wo_i, dimension_numbers=dn,
                             preferred_element_type=jnp.float32)
        acc_a = ca if acc_a is None else acc_a + ca
        acc_b = cb if acc_b is None else acc_b + cb
    out_ref[0] = acc_a + bo_ref[...]
    out_ref[1] = acc_b + bo_ref[...]


def _final(parts3, g, wo, bo, blkp=1024):
    nh, half, hd = parts3.shape
    d = wo.shape[0]
    nblk = half // blkp
    return pl.pallas_call(
        _final_body,
        grid=(nblk,),
        in_specs=[
            pl.BlockSpec((nh, blkp, hd), lambda t: (0, t, 0)),
            pl.BlockSpec((blkp, g.shape[1]), lambda t: (t, 0)),
            pl.BlockSpec((blkp, g.shape[1]), lambda t, n=nblk: (t + n, 0)),
            pl.BlockSpec(wo.shape, lambda t: (0, 0)),
            pl.BlockSpec((1, d), lambda t: (0, 0)),
        ],
        out_specs=pl.BlockSpec((2, blkp, d), lambda t: (0, t, 0)),
        out_shape=jax.ShapeDtypeStruct((2, half, d), jnp.float32),
    )(parts3, g, g, wo, bo)


def kernel(input_ids, hidden_state, tables, W_gate, b_gate, W_out, b_out):
    b, s = input_ids.shape
    nh, buckets, hd = tables.shape
    d = hidden_state.shape[-1]
    bs = b * s

    ids = input_ids.astype(jnp.int32)
    idx_flat, tables_u = _prep(ids, tables.reshape(nh * buckets, hd), nh, buckets)

    half = bs // 2
    parts = _make_sc_gather(nh * buckets, hd, nh * bs, half)(tables_u, idx_flat)
    parts3 = parts.reshape(nh, half, hd)

    hid2 = hidden_state.reshape(bs, d)
    bg = b_gate.reshape(1, nh)
    bo = b_out.reshape(1, d)

    g = _gates(hid2, W_gate, bg)
    out = _final(parts3, g, W_out, bo)
    return out.reshape(b, s, d)
